# Initial kernel scaffold; baseline (speedup 1.0000x reference)
import jax
import jax.numpy as jnp
from jax.experimental import pallas as pl


def kernel(x, pos, norm, batch, W1, b1, W2, b2, W3, b3):
    raise NotImplementedError("write your pallas kernel here")


# speedup vs baseline: 10.5393x; 10.5393x over previous
import functools

import jax
import jax.numpy as jnp
from jax import lax
from jax.experimental import pallas as pl
from jax.experimental.pallas import tpu as pltpu
from jax.experimental.pallas import tpu_sc as plsc

_NC, _NS, _L = 2, 16, 16
_NW = _NC * _NS
_K = 32
_R2 = 0.04000000000000001
_TW = 128


def _fps_body(pos3_ref, out_ref, dist_ref):
    RB = dist_ref.shape[0]
    SB = out_ref.shape[0]
    posx = pos3_ref[0]
    posy = pos3_ref[1]
    posz = pos3_ref[2]
    lin = (lax.broadcasted_iota(jnp.int32, (RB, 128), 0) * 128
           + lax.broadcasted_iota(jnp.int32, (RB, 128), 1))
    lin_o = (lax.broadcasted_iota(jnp.int32, (SB, 128), 0) * 128
             + lax.broadcasted_iota(jnp.int32, (SB, 128), 1))
    dist_ref[...] = jnp.full((RB, 128), jnp.inf, jnp.float32)
    out_ref[...] = jnp.zeros((SB, 128), jnp.int32)

    def body(i, last):
        lm = lin == last
        px = jnp.sum(jnp.where(lm, posx, 0.0))
        py = jnp.sum(jnp.where(lm, posy, 0.0))
        pz = jnp.sum(jnp.where(lm, posz, 0.0))
        dx = posx - px
        dy = posy - py
        dz = posz - pz
        d = dx * dx + dy * dy + dz * dz
        dm = jnp.minimum(dist_ref[...], d)
        dist_ref[...] = dm
        m = jnp.max(dm)
        li = jnp.min(jnp.where(dm == m, lin, jnp.int32(2147483647)))
        out_ref[...] = jnp.where(lin_o == i, li, out_ref[...])
        return li

    lax.fori_loop(1, SB * 128, body, jnp.int32(0))


def _fps_pallas(pos, n_samples):
    N = pos.shape[0]
    RB = N // 128
    SB = n_samples // 128
    pos3 = pos.T.reshape(3, RB, 128)
    out = pl.pallas_call(
        _fps_body,
        out_shape=jax.ShapeDtypeStruct((SB, 128), jnp.int32),
        scratch_shapes=[pltpu.VMEM((RB, 128), jnp.float32)],
    )(pos3)
    return out.reshape(n_samples)


def _prep_body(x_ref, pn_ref, w_ref, t_ref):
    blk = x_ref.shape[0]
    xb = x_ref[...].astype(jnp.bfloat16)
    w = w_ref[...].astype(jnp.bfloat16)
    xe = lax.dot_general(xb, w, (((1,), (0,)), ((), ())),
                         preferred_element_type=jnp.float32)
    t_ref[...] = jnp.concatenate(
        [xe, pn_ref[...], jnp.zeros((blk, _TW - 72), jnp.float32)], axis=1)


def _prep_pallas(x, pn, w1a):
    N, DF = x.shape
    blk = 2048
    grid = N // blk
    return pl.pallas_call(
        _prep_body,
        grid=(grid,),
        in_specs=[
            pl.BlockSpec((blk, DF), lambda i: (i, 0)),
            pl.BlockSpec((blk, 8), lambda i: (i, 0)),
            pl.BlockSpec((DF, 64), lambda i: (0, 0)),
        ],
        out_specs=pl.BlockSpec((blk, _TW), lambda i: (i, 0)),
        out_shape=jax.ShapeDtypeStruct((N, _TW), jnp.float32),
    )(x, pn, w1a)


def _bf16r(v):
    u = lax.bitcast_convert_type(v, jnp.int32)
    lsb = lax.shift_right_logical(u, 16) & 1
    r = (u + 32767 + lsb) & jnp.int32(-65536)
    return lax.bitcast_convert_type(r, jnp.float32)


def _radius_sc(posx, posy, posz, idx):
    N = posx.shape[0]
    S = idx.shape[0]
    QW = S // _NW
    CH = N // _L
    mesh = plsc.VectorSubcoreMesh(core_axis_name="c", subcore_axis_name="s")

    @functools.partial(
        pl.kernel,
        out_type=(jax.ShapeDtypeStruct((_K, S), jnp.int32),
                  jax.ShapeDtypeStruct((S * _K,), jnp.int32)),
        mesh=mesh,
        compiler_params=pltpu.CompilerParams(needs_layout_passes=False),
        scratch_types=[
            pltpu.VMEM((N + _L,), jnp.float32),
            pltpu.VMEM((N + _L,), jnp.float32),
            pltpu.VMEM((N + _L,), jnp.float32),
            pltpu.VMEM((N + _L,), jnp.float32),
            pltpu.VMEM((QW + _L,), jnp.int32),
            pltpu.VMEM((QW * 64,), jnp.int32),
            pltpu.VMEM((QW * _K,), jnp.int32),
            pltpu.VMEM((_K, QW), jnp.int32),
        ],
    )
    def k(posx_h, posy_h, posz_h, idx_h, nbrt_h, valid_h,
          px_v, py_v, pz_v, sp_v, idx_v, nbr_v, val_v, nbrt_v):
        wid = lax.axis_index("s") * _NC + lax.axis_index("c")
        pltpu.sync_copy(posx_h, px_v.at[pl.ds(0, N)])
        pltpu.sync_copy(posy_h, py_v.at[pl.ds(0, N)])
        pltpu.sync_copy(posz_h, pz_v.at[pl.ds(0, N)])
        pltpu.sync_copy(idx_h.at[pl.ds(wid * QW, QW)],
                        idx_v.at[pl.ds(0, QW)])
        iota = lax.iota(jnp.int32, _L)

        def pre(c, _):
            s = pl.ds(c * _L, _L)
            px = px_v[s]
            py = py_v[s]
            pz = pz_v[s]
            sp_v[s] = (px * px + py * py) + pz * pz
            px_v[s] = _bf16r(px)
            py_v[s] = _bf16r(py)
            pz_v[s] = _bf16r(pz)
            return 0

        lax.fori_loop(0, CH, pre, 0)

        def per_query(q, _):
            qi = idx_v[pl.ds(q, _L)][0]
            qxb = jnp.full((_L,), px_v[pl.ds(qi, _L)][0])
            qyb = jnp.full((_L,), py_v[pl.ds(qi, _L)][0])
            qzb = jnp.full((_L,), pz_v[pl.ds(qi, _L)][0])
            sqb = jnp.full((_L,), sp_v[pl.ds(qi, _L)][0])

            def cond(st):
                c, cnt = st
                return jnp.logical_and(cnt < _K, c < CH)

            def body(st):
                c, cnt = st
                s = pl.ds(c * _L, _L)
                px = px_v[s]
                py = py_v[s]
                pz = pz_v[s]
                sp = sp_v[s]
                dot = (qxb * px + qyb * py) + qzb * pz
                d2 = (sqb + sp) - 2.0 * dot
                m = d2 <= _R2
                pid = c * _L + iota
                plsc.store_compressed(
                    nbr_v.at[pl.ds(q * 64 + cnt, _L)], pid, mask=m)
                pc = plsc.all_reduce_population_count(m)
                return c + 1, cnt + pc[0]

            _, cnt = lax.while_loop(cond, body,
                                    (jnp.int32(0), jnp.int32(0)))
            cntq = jnp.minimum(cnt, _K)
            n0 = jnp.where(iota < cntq, nbr_v[pl.ds(q * 64, _L)], 0)
            n1 = jnp.where(iota + _L < cntq,
                           nbr_v[pl.ds(q * 64 + _L, _L)], 0)
            val_v[pl.ds(q * _K, _L)] = (iota < cntq).astype(jnp.int32)
            val_v[pl.ds(q * _K + _L, _L)] = \
                (iota + _L < cntq).astype(jnp.int32)
            qf = jnp.full((_L,), q)
            plsc.store_scatter(nbrt_v, [iota, qf], n0)
            plsc.store_scatter(nbrt_v, [iota + _L, qf], n1)
            return 0

        lax.fori_loop(0, QW, per_query, 0)
        pltpu.sync_copy(val_v, valid_h.at[pl.ds(wid * QW * _K, QW * _K)])

        def wr(j, _):
            pltpu.sync_copy(nbrt_v.at[j], nbrt_h.at[j, pl.ds(wid * QW, QW)])
            return 0

        lax.fori_loop(0, _K, wr, 0)

    return k(posx, posy, posz, idx)


def _gather_sc(table, nbrt, idx):
    N = table.shape[0]
    S = idx.shape[0]
    QW = S // _NW
    mesh = plsc.VectorSubcoreMesh(core_axis_name="c", subcore_axis_name="s")

    @functools.partial(
        pl.kernel,
        out_type=(jax.ShapeDtypeStruct((S * _K, _TW), jnp.float32),
                  jax.ShapeDtypeStruct((S, _TW), jnp.float32)),
        mesh=mesh,
        scratch_types=[
            pltpu.VMEM((QW,), jnp.int32),
            pltpu.VMEM((QW, _TW), jnp.float32),
            pltpu.SemaphoreType.DMA,
        ],
    )
    def k(t_h, nbrt_h, idx_h, e_h, qe_h, idx_v, rows_v, sem):
        wid = lax.axis_index("s") * _NC + lax.axis_index("c")
        pltpu.sync_copy(idx_h.at[pl.ds(wid * QW, QW)], idx_v)
        pltpu.async_copy(t_h.at[idx_v], rows_v, sem).wait()
        pltpu.sync_copy(rows_v, qe_h.at[pl.ds(wid * QW, QW)])

        def per_j(j, _):
            pltpu.sync_copy(nbrt_h.at[j, pl.ds(wid * QW, QW)], idx_v)
            pltpu.async_copy(t_h.at[idx_v], rows_v, sem).wait()
            pltpu.sync_copy(
                rows_v, e_h.at[pl.ds(wid * _K * QW + j * QW, QW)])
            return 0

        lax.fori_loop(0, _K, per_j, 0)

    return k(table, nbrt, idx)


def _angle3(ax, ay, az, bx, by, bz):
    cx = ay * bz - az * by
    cy = az * bx - ax * bz
    cz = ax * by - ay * bx
    cn = jnp.sqrt((cx * cx + cy * cy) + cz * cz)
    d = (ax * bx + ay * by) + az * bz
    return jnp.arctan2(cn, d)


def _mlp_body(e_ref, qe_ref, val_ref, w1b_ref, b1_ref, w2_ref, b2_ref,
              w3_ref, b3_ref, out_ref):
    BQ = qe_ref.shape[0]
    qe = qe_ref[...]
    val = val_ref[...]
    w1b = w1b_ref[...].astype(jnp.bfloat16)
    w2 = w2_ref[...].astype(jnp.bfloat16)
    w3 = w3_ref[...].astype(jnp.bfloat16)
    b1 = b1_ref[...]
    b2 = b2_ref[...]
    b3 = b3_ref[...]
    qpx = qe[:, 64:65]
    qpy = qe[:, 65:66]
    qpz = qe[:, 66:67]
    qnx = qe[:, 67:68]
    qny = qe[:, 68:69]
    qnz = qe[:, 69:70]
    acc = jnp.full((BQ, 128), -jnp.inf, jnp.float32)
    dn = (((1,), (0,)), ((), ()))
    for j in range(_K):
        ej = e_ref[pl.ds(j * BQ, BQ), :]
        xe = ej[:, 0:64]
        dx = ej[:, 64:65] - qpx
        dy = ej[:, 65:66] - qpy
        dz = ej[:, 66:67] - qpz
        njx = ej[:, 67:68]
        njy = ej[:, 68:69]
        njz = ej[:, 69:70]
        f1 = jnp.sqrt((dx * dx + dy * dy) + dz * dz)
        f2 = _angle3(qnx, qny, qnz, dx, dy, dz)
        f3 = _angle3(njx, njy, njz, dx, dy, dz)
        f4 = _angle3(qnx, qny, qnz, njx, njy, njz)
        ppf = jnp.concatenate([f1, f2, f3, f4], axis=1).astype(jnp.bfloat16)
        dot4 = lax.dot_general(ppf, w1b, dn,
                               preferred_element_type=jnp.float32)
        h1 = jax.nn.relu((xe + dot4) + b1)
        h2 = jax.nn.relu(
            lax.dot_general(h1.astype(jnp.bfloat16), w2, dn,
                            preferred_element_type=jnp.float32) + b2)
        h3 = lax.dot_general(h2.astype(jnp.bfloat16), w3, dn,
                             preferred_element_type=jnp.float32) + b3
        vj = val[:, j:j + 1] > 0
        acc = jnp.maximum(acc, jnp.where(vj, h3, -jnp.inf))
    out_ref[...] = acc


def _mlp_pallas(e, qe, valid, w1b, b1, w2, b2, w3, b3):
    S = qe.shape[0]
    BQ = 128
    grid = S // BQ
    return pl.pallas_call(
        _mlp_body,
        grid=(grid,),
        in_specs=[
            pl.BlockSpec((_K * BQ, _TW), lambda i: (i, 0)),
            pl.BlockSpec((BQ, _TW), lambda i: (i, 0)),
            pl.BlockSpec((BQ, _K), lambda i: (i, 0)),
            pl.BlockSpec((4, 64), lambda i: (0, 0)),
            pl.BlockSpec((1, 64), lambda i: (0, 0)),
            pl.BlockSpec((64, 64), lambda i: (0, 0)),
            pl.BlockSpec((1, 64), lambda i: (0, 0)),
            pl.BlockSpec((64, 128), lambda i: (0, 0)),
            pl.BlockSpec((1, 128), lambda i: (0, 0)),
        ],
        out_specs=pl.BlockSpec((BQ, 128), lambda i: (i, 0)),
        out_shape=jax.ShapeDtypeStruct((S, 128), jnp.float32),
    )(e, qe, valid, w1b, b1, w2, b2, w3, b3)


def kernel(x, pos, norm, batch, W1, b1, W2, b2, W3, b3):
    N = pos.shape[0]
    S = N // 4

    idx = _fps_pallas(pos, S)

    batchf = lax.bitcast_convert_type(batch, jnp.float32).reshape(N, 1)
    pn = jnp.concatenate(
        [pos, norm, batchf, jnp.zeros((N, 1), jnp.float32)], axis=1)
    table = _prep_pallas(x, pn, W1[:64])

    post = pos.T
    posx, posy, posz = post[0], post[1], post[2]
    nbrt, valid = _radius_sc(posx, posy, posz, idx)

    e, qe = _gather_sc(table, nbrt, idx)

    out = _mlp_pallas(e, qe, valid.reshape(S, _K), W1[64:68],
                      b1.reshape(1, 64), W2, b2.reshape(1, 64),
                      W3, b3.reshape(1, 128))

    pos_q = qe[:, 64:67]
    norm_q = qe[:, 67:70]
    batch_q = lax.bitcast_convert_type(qe[:, 70], jnp.int32)
    return (out, pos_q, norm_q, batch_q)
